# baseline (speedup 1.0000x reference)
import functools

import jax
import jax.numpy as jnp
from jax import lax
from jax.experimental import pallas as pl
from jax.experimental.pallas import tpu as pltpu
from jax.experimental.pallas import tpu_sc as plsc

_NC, _NS = 2, 16
_NW = _NC * _NS
_NSLICE = 2
_R = 1024


def _sc_sum_body(atoms_hbm, bonds_hbm, out_hbm,
                 at0, at1, bd0, bd1, st0, st1, sin0, sin1, sout0, sout1,
                 *, base, spw, A, F, D):
    wid = lax.axis_index("s") * _NC + lax.axis_index("c")
    s0 = base + wid * spw
    o0 = wid * spw
    nf = F // 16
    ats = (at0, at1)
    bds = (bd0, bd1)
    sts = (st0, st1)
    sins = (sin0, sin1)
    souts = (sout0, sout1)
    zero16 = jnp.zeros((16,), jnp.float32)
    for j in range(nf):
        at0[A, pl.ds(16 * j, 16)] = zero16
        at1[A, pl.ds(16 * j, 16)] = zero16

    def issue_in(si, p):
        pltpu.async_copy(atoms_hbm.at[s0 + si], ats[p].at[pl.ds(0, A)], sins[p])
        pltpu.async_copy(bonds_hbm.at[s0 + si], bds[p], sins[p])

    def wait_in(si, p):
        pltpu.make_async_copy(atoms_hbm.at[s0 + si],
                              ats[p].at[pl.ds(0, A)], sins[p]).wait()
        pltpu.make_async_copy(bonds_hbm.at[s0 + si], bds[p], sins[p]).wait()

    def wait_out(si, p):
        pltpu.make_async_copy(sts[p], out_hbm.at[o0 + si], souts[p]).wait()

    issue_in(0, 0)

    def pair_body(i, carry):
        for par in range(2):
            si = 2 * i + par
            wait_in(si, par)

            @pl.when(si + 1 < spw)
            def _():
                issue_in(si + 1, 1 - par)

            @pl.when(si >= 2)
            def _():
                wait_out(si - 2, par)

            at_v = ats[par]
            bd_v = bds[par]
            st_v = sts[par]

            @plsc.parallel_loop(0, A, unroll=4)
            def per_atom(a):
                brow = bd_v[a, pl.ds(0, D)]
                rows = jnp.where(brow < 0, A, brow)
                accs = [at_v[a, pl.ds(16 * j, 16)] for j in range(nf)]
                for d in range(D):
                    r = rows[d]
                    for j in range(nf):
                        accs[j] = accs[j] + at_v[r, pl.ds(16 * j, 16)]
                for j in range(nf):
                    st_v[a, pl.ds(16 * j, 16)] = accs[j]

            pltpu.async_copy(st_v, out_hbm.at[o0 + si], souts[par])
        return carry

    lax.fori_loop(0, spw // 2, pair_body, 0)
    wait_out(spw - 2, 0)
    wait_out(spw - 1, 1)


def _dense_body(sum_ref, bonds_ref, wp_ref, b_ref, out_ref, *, R, D, C):
    summed = sum_ref[...]
    bb = bonds_ref[...]
    full = lax.dot_general(summed, wp_ref[...], (((1,), (0,)), ((), ())),
                           preferred_element_type=jnp.float32)
    full = full + b_ref[...]
    deg = jnp.sum((bb != -1).astype(jnp.int32), axis=1, keepdims=True)
    acc = jnp.zeros((R, C), jnp.float32)
    for d in range(D):
        acc = acc + jnp.where(deg == d, full[:, d * C:(d + 1) * C], 0.0)
    out_ref[...] = acc


def kernel(atoms, bonds, Ws, bs):
    S, A, F = atoms.shape
    D, _, C = Ws.shape
    wp = jnp.transpose(Ws, (1, 0, 2)).reshape(F, D * C)
    br = bs.reshape(1, D * C)
    bonds32 = bonds.astype(jnp.int32)
    bonds_rows = bonds32.reshape(S * A, D)

    mesh = plsc.VectorSubcoreMesh(core_axis_name="c", subcore_axis_name="s",
                                  num_cores=_NC, num_subcores=_NS)
    ssl = S // _NSLICE
    spw = ssl // _NW
    outs = []
    for k in range(_NSLICE):
        summed_k = pl.kernel(
            functools.partial(_sc_sum_body, base=k * ssl, spw=spw,
                              A=A, F=F, D=D),
            out_type=jax.ShapeDtypeStruct((ssl, A, F), jnp.float32),
            mesh=mesh,
            compiler_params=pltpu.CompilerParams(needs_layout_passes=False),
            scratch_types=[
                pltpu.VMEM((A + 1, F), jnp.float32),
                pltpu.VMEM((A + 1, F), jnp.float32),
                pltpu.VMEM((A, D), jnp.int32),
                pltpu.VMEM((A, D), jnp.int32),
                pltpu.VMEM((A, F), jnp.float32),
                pltpu.VMEM((A, F), jnp.float32),
                pltpu.SemaphoreType.DMA,
                pltpu.SemaphoreType.DMA,
                pltpu.SemaphoreType.DMA,
                pltpu.SemaphoreType.DMA,
            ],
        )(atoms, bonds32)

        rbase = k * ssl * A // _R
        out_k = pl.pallas_call(
            functools.partial(_dense_body, R=_R, D=D, C=C),
            grid=(ssl * A // _R,),
            in_specs=[
                pl.BlockSpec((_R, F), lambda r: (r, 0)),
                pl.BlockSpec((_R, D), lambda r, rb=rbase: (rb + r, 0)),
                pl.BlockSpec((F, D * C), lambda r: (0, 0)),
                pl.BlockSpec((1, D * C), lambda r: (0, 0)),
            ],
            out_specs=pl.BlockSpec((_R, C), lambda r: (r, 0)),
            out_shape=jax.ShapeDtypeStruct((ssl * A, C), jnp.float32),
        )(summed_k.reshape(ssl * A, F), bonds_rows, wp, br)
        outs.append(out_k.reshape(ssl, A, C))
    return jnp.concatenate(outs, axis=0)


# speedup vs baseline: 1.2595x; 1.2595x over previous
import functools

import jax
import jax.numpy as jnp
from jax import lax
from jax.experimental import pallas as pl
from jax.experimental.pallas import tpu as pltpu
from jax.experimental.pallas import tpu_sc as plsc

_NC, _NS = 2, 16
_NW = _NC * _NS
_NSLICE = 2
_R = 1024


def _sc_sum_body(atoms_hbm, bonds_hbm, out_hbm,
                 at0, at1, bd0, bd1, st0, st1, sin0, sin1, sout0, sout1,
                 *, base, spw, A, F, D):
    wid = lax.axis_index("s") * _NC + lax.axis_index("c")
    s0 = base + wid * spw
    o0 = wid * spw
    nf = F // 16
    ats = (at0, at1)
    bds = (bd0, bd1)
    sts = (st0, st1)
    sins = (sin0, sin1)
    souts = (sout0, sout1)
    zero16 = jnp.zeros((16,), jnp.float32)
    for j in range(nf):
        at0[A, pl.ds(16 * j, 16)] = zero16
        at1[A, pl.ds(16 * j, 16)] = zero16

    def issue_in(si, p):
        pltpu.async_copy(atoms_hbm.at[s0 + si], ats[p].at[pl.ds(0, A)], sins[p])
        pltpu.async_copy(bonds_hbm.at[s0 + si], bds[p], sins[p])

    def wait_in(si, p):
        pltpu.make_async_copy(atoms_hbm.at[s0 + si],
                              ats[p].at[pl.ds(0, A)], sins[p]).wait()
        pltpu.make_async_copy(bonds_hbm.at[s0 + si], bds[p], sins[p]).wait()

    def wait_out(si, p):
        pltpu.make_async_copy(sts[p], out_hbm.at[o0 + si], souts[p]).wait()

    issue_in(0, 0)

    def pair_body(i, carry):
        for par in range(2):
            si = 2 * i + par
            wait_in(si, par)

            @pl.when(si + 1 < spw)
            def _():
                issue_in(si + 1, 1 - par)

            @pl.when(si >= 2)
            def _():
                wait_out(si - 2, par)

            at_v = ats[par]
            bd_v = bds[par]
            st_v = sts[par]

            @plsc.parallel_loop(0, A, unroll=2)
            def per_atom(a):
                brow = bd_v[a, pl.ds(0, D)]
                rows = jnp.where(brow < 0, A, brow)
                accs = [at_v[a, pl.ds(16 * j, 16)] for j in range(nf)]
                for d in range(D):
                    r = rows[d]
                    for j in range(nf):
                        accs[j] = accs[j] + at_v[r, pl.ds(16 * j, 16)]
                for j in range(nf):
                    st_v[a, pl.ds(16 * j, 16)] = accs[j]

            pltpu.async_copy(st_v, out_hbm.at[o0 + si], souts[par])
        return carry

    lax.fori_loop(0, spw // 2, pair_body, 0)
    wait_out(spw - 2, 0)
    wait_out(spw - 1, 1)


def _dense_body(sum_ref, bonds_ref, wp_ref, b_ref, out_ref, *, R, D, C):
    summed = sum_ref[...]
    bb = bonds_ref[...]
    full = lax.dot_general(summed, wp_ref[...], (((1,), (0,)), ((), ())),
                           preferred_element_type=jnp.float32)
    full = full + b_ref[...]
    deg = jnp.sum((bb != -1).astype(jnp.int32), axis=1, keepdims=True)
    acc = jnp.zeros((R, C), jnp.float32)
    for d in range(D):
        acc = acc + jnp.where(deg == d, full[:, d * C:(d + 1) * C], 0.0)
    out_ref[...] = acc


def kernel(atoms, bonds, Ws, bs):
    S, A, F = atoms.shape
    D, _, C = Ws.shape
    wp = jnp.transpose(Ws, (1, 0, 2)).reshape(F, D * C)
    br = bs.reshape(1, D * C)
    bonds32 = bonds.astype(jnp.int32)
    bonds_rows = bonds32.reshape(S * A, D)

    mesh = plsc.VectorSubcoreMesh(core_axis_name="c", subcore_axis_name="s",
                                  num_cores=_NC, num_subcores=_NS)
    ssl = S // _NSLICE
    spw = ssl // _NW
    outs = []
    for k in range(_NSLICE):
        summed_k = pl.kernel(
            functools.partial(_sc_sum_body, base=k * ssl, spw=spw,
                              A=A, F=F, D=D),
            out_type=jax.ShapeDtypeStruct((ssl, A, F), jnp.float32),
            mesh=mesh,
            compiler_params=pltpu.CompilerParams(needs_layout_passes=False),
            scratch_types=[
                pltpu.VMEM((A + 1, F), jnp.float32),
                pltpu.VMEM((A + 1, F), jnp.float32),
                pltpu.VMEM((A, D), jnp.int32),
                pltpu.VMEM((A, D), jnp.int32),
                pltpu.VMEM((A, F), jnp.float32),
                pltpu.VMEM((A, F), jnp.float32),
                pltpu.SemaphoreType.DMA,
                pltpu.SemaphoreType.DMA,
                pltpu.SemaphoreType.DMA,
                pltpu.SemaphoreType.DMA,
            ],
        )(atoms, bonds32)

        rbase = k * ssl * A // _R
        out_k = pl.pallas_call(
            functools.partial(_dense_body, R=_R, D=D, C=C),
            grid=(ssl * A // _R,),
            in_specs=[
                pl.BlockSpec((_R, F), lambda r: (r, 0)),
                pl.BlockSpec((_R, D), lambda r, rb=rbase: (rb + r, 0)),
                pl.BlockSpec((F, D * C), lambda r: (0, 0)),
                pl.BlockSpec((1, D * C), lambda r: (0, 0)),
            ],
            out_specs=pl.BlockSpec((_R, C), lambda r: (r, 0)),
            out_shape=jax.ShapeDtypeStruct((ssl * A, C), jnp.float32),
        )(summed_k.reshape(ssl * A, F), bonds_rows, wp, br)
        outs.append(out_k.reshape(ssl, A, C))
    return jnp.concatenate(outs, axis=0)


# speedup vs baseline: 1.2604x; 1.0007x over previous
import functools

import jax
import jax.numpy as jnp
from jax import lax
from jax.experimental import pallas as pl
from jax.experimental.pallas import tpu as pltpu
from jax.experimental.pallas import tpu_sc as plsc

_NC, _NS = 2, 16
_NW = _NC * _NS
_NSLICE = 2
_R = 1024


def _sc_sum_body(atoms_hbm, bonds_hbm, out_hbm,
                 at0, at1, bd0, bd1, st0, st1, sin0, sin1, sout0, sout1,
                 *, base, spw, A, F, D):
    wid = lax.axis_index("s") * _NC + lax.axis_index("c")
    s0 = base + wid * spw
    o0 = wid * spw
    nf = F // 16
    ats = (at0, at1)
    bds = (bd0, bd1)
    sts = (st0, st1)
    sins = (sin0, sin1)
    souts = (sout0, sout1)
    zero16 = jnp.zeros((16,), jnp.float32)
    for j in range(nf):
        at0[A, pl.ds(16 * j, 16)] = zero16
        at1[A, pl.ds(16 * j, 16)] = zero16

    def issue_in(si, p):
        pltpu.async_copy(atoms_hbm.at[s0 + si], ats[p].at[pl.ds(0, A)], sins[p])
        pltpu.async_copy(bonds_hbm.at[s0 + si], bds[p], sins[p])

    def wait_in(si, p):
        pltpu.make_async_copy(atoms_hbm.at[s0 + si],
                              ats[p].at[pl.ds(0, A)], sins[p]).wait()
        pltpu.make_async_copy(bonds_hbm.at[s0 + si], bds[p], sins[p]).wait()

    def wait_out(si, p):
        pltpu.make_async_copy(sts[p], out_hbm.at[o0 + si], souts[p]).wait()

    issue_in(0, 0)

    def pair_body(i, carry):
        for par in range(2):
            si = 2 * i + par
            wait_in(si, par)

            @pl.when(si + 1 < spw)
            def _():
                issue_in(si + 1, 1 - par)

            @pl.when(si >= 2)
            def _():
                wait_out(si - 2, par)

            at_v = ats[par]
            bd_v = bds[par]
            st_v = sts[par]

            @plsc.parallel_loop(0, A, unroll=2)
            def per_atom(a):
                brow = bd_v[a, pl.ds(0, D)]
                rows = jnp.where(brow < 0, A, brow)
                accs = [at_v[a, pl.ds(16 * j, 16)] for j in range(nf)]
                for d in range(D):
                    r = rows[d]
                    for j in range(nf):
                        accs[j] = accs[j] + at_v[r, pl.ds(16 * j, 16)]
                for j in range(nf):
                    st_v[a, pl.ds(16 * j, 16)] = accs[j]

            pltpu.async_copy(st_v, out_hbm.at[o0 + si], souts[par])
        return carry

    lax.fori_loop(0, spw // 2, pair_body, 0)
    wait_out(spw - 2, 0)
    wait_out(spw - 1, 1)


def _dense_body(sum_ref, bonds_ref, wp_ref, b_ref, out_ref, *, R, D, C):
    summed = sum_ref[...].astype(jnp.bfloat16)
    bb = bonds_ref[...]
    full = lax.dot_general(summed, wp_ref[...], (((1,), (0,)), ((), ())),
                           preferred_element_type=jnp.float32)
    full = full + b_ref[...]
    deg = jnp.sum((bb != -1).astype(jnp.int32), axis=1, keepdims=True)
    acc = jnp.zeros((R, C), jnp.float32)
    for d in range(D):
        acc = acc + jnp.where(deg == d, full[:, d * C:(d + 1) * C], 0.0)
    out_ref[...] = acc


def kernel(atoms, bonds, Ws, bs):
    S, A, F = atoms.shape
    D, _, C = Ws.shape
    wp = jnp.transpose(Ws, (1, 0, 2)).reshape(F, D * C)
    wp_bf = wp.astype(jnp.bfloat16)
    br = bs.reshape(1, D * C)
    bonds32 = bonds.astype(jnp.int32)
    bonds_rows = bonds32.reshape(S * A, D)

    mesh = plsc.VectorSubcoreMesh(core_axis_name="c", subcore_axis_name="s",
                                  num_cores=_NC, num_subcores=_NS)
    ssl = S // _NSLICE
    spw = ssl // _NW
    outs = []
    for k in range(_NSLICE):
        summed_k = pl.kernel(
            functools.partial(_sc_sum_body, base=k * ssl, spw=spw,
                              A=A, F=F, D=D),
            out_type=jax.ShapeDtypeStruct((ssl, A, F), jnp.float32),
            mesh=mesh,
            compiler_params=pltpu.CompilerParams(needs_layout_passes=False),
            scratch_types=[
                pltpu.VMEM((A + 1, F), jnp.float32),
                pltpu.VMEM((A + 1, F), jnp.float32),
                pltpu.VMEM((A, D), jnp.int32),
                pltpu.VMEM((A, D), jnp.int32),
                pltpu.VMEM((A, F), jnp.float32),
                pltpu.VMEM((A, F), jnp.float32),
                pltpu.SemaphoreType.DMA,
                pltpu.SemaphoreType.DMA,
                pltpu.SemaphoreType.DMA,
                pltpu.SemaphoreType.DMA,
            ],
        )(atoms, bonds32)

        rbase = k * ssl * A // _R
        out_k = pl.pallas_call(
            functools.partial(_dense_body, R=_R, D=D, C=C),
            grid=(ssl * A // _R,),
            in_specs=[
                pl.BlockSpec((_R, F), lambda r: (r, 0)),
                pl.BlockSpec((_R, D), lambda r, rb=rbase: (rb + r, 0)),
                pl.BlockSpec((F, D * C), lambda r: (0, 0)),
                pl.BlockSpec((1, D * C), lambda r: (0, 0)),
            ],
            out_specs=pl.BlockSpec((_R, C), lambda r: (r, 0)),
            out_shape=jax.ShapeDtypeStruct((ssl * A, C), jnp.float32),
        )(summed_k.reshape(ssl * A, F), bonds_rows, wp_bf, br)
        outs.append(out_k.reshape(ssl, A, C))
    return jnp.concatenate(outs, axis=0)
